# baseline (speedup 1.0000x reference)
import functools

import jax
import jax.numpy as jnp
import numpy as np
from jax import lax
from jax.experimental import pallas as pl
from jax.experimental.pallas import tpu as pltpu
from jax.experimental.pallas import tpu_sc as plsc

_C = 32
_NTRIL = _C * (_C + 1) // 2
_M = 256 * 32 * 8
_LANES = 16

_NC = 2
_NS = 16
_NW = _NC * _NS
_M_PER_W = _M // _NW

_G = 16
_CHUNKS = _M_PER_W // _G
_VECS = _G * _NTRIL // _LANES


def _build_idx() -> np.ndarray:
    row, col = np.tril_indices(_C)
    lin = (row * _C + col).astype(np.int32)
    m = np.arange(_G, dtype=np.int32)[:, None]
    return (m * (_C * _C) + lin[None, :]).reshape(-1)


_IDX_HOST = _build_idx()


def _body(in_hbm, idx_hbm, out_hbm, idx_v, in_v, out_v):
    wid = lax.axis_index("s") * _NC + lax.axis_index("c")
    base = wid * _M_PER_W
    pltpu.sync_copy(idx_hbm, idx_v)

    def chunk(i, carry):
        m0 = base + i * _G
        pltpu.sync_copy(in_hbm.at[pl.ds(m0 * (_C * _C), _G * _C * _C)], in_v)

        def compact(k, c):
            idx = idx_v[pl.ds(k * _LANES, _LANES)]
            out_v[pl.ds(k * _LANES, _LANES)] = plsc.load_gather(in_v, [idx])
            return c

        lax.fori_loop(0, _VECS, compact, 0, unroll=4)
        pltpu.sync_copy(out_v, out_hbm.at[pl.ds(m0 * _NTRIL, _G * _NTRIL)])
        return carry

    lax.fori_loop(0, _CHUNKS, chunk, 0)


@jax.jit
def _spd_vectorize(flat_in, idx):
    mesh = plsc.VectorSubcoreMesh(core_axis_name="c", subcore_axis_name="s")
    return pl.kernel(
        _body,
        mesh=mesh,
        out_type=jax.ShapeDtypeStruct((_M * _NTRIL,), jnp.float32),
        scratch_types=[
            pltpu.VMEM((_G * _NTRIL,), jnp.int32),
            pltpu.VMEM((_G * _C * _C,), jnp.float32),
            pltpu.VMEM((_G * _NTRIL,), jnp.float32),
        ],
        compiler_params=pltpu.CompilerParams(needs_layout_passes=False),
    )(flat_in, idx)


def kernel(inputs):
    flat = inputs.reshape(-1)
    idx = jnp.asarray(_IDX_HOST)
    out = _spd_vectorize(flat, idx)
    return out.reshape(256, 32, 8, _NTRIL)


# speedup vs baseline: 1.4649x; 1.4649x over previous
import functools

import jax
import jax.numpy as jnp
import numpy as np
from jax import lax
from jax.experimental import pallas as pl
from jax.experimental.pallas import tpu as pltpu
from jax.experimental.pallas import tpu_sc as plsc

_C = 32
_NTRIL = _C * (_C + 1) // 2
_M = 256 * 32 * 8
_LANES = 16

_NC = 2
_NS = 16
_NW = _NC * _NS
_M_PER_W = _M // _NW

_G = 16
_CHUNKS = _M_PER_W // _G
_VECS = _G * _NTRIL // _LANES


def _build_idx() -> np.ndarray:
    row, col = np.tril_indices(_C)
    lin = (row * _C + col).astype(np.int32)
    m = np.arange(_G, dtype=np.int32)[:, None]
    return (m * (_C * _C) + lin[None, :]).reshape(-1)


_IDX_HOST = _build_idx()


def _body(in_hbm, idx_hbm, out_hbm, idx_v, in_v0, in_v1, out_v0, out_v1,
          in_sem0, in_sem1, out_sem0, out_sem1):
    wid = lax.axis_index("s") * _NC + lax.axis_index("c")
    base = wid * _M_PER_W
    pltpu.sync_copy(idx_hbm, idx_v)

    in_bufs = (in_v0, in_v1)
    out_bufs = (out_v0, out_v1)
    in_sems = (in_sem0, in_sem1)
    out_sems = (out_sem0, out_sem1)

    def in_copy(g, b):
        m0 = base + g * _G
        return pltpu.make_async_copy(
            in_hbm.at[pl.ds(m0 * (_C * _C), _G * _C * _C)],
            in_bufs[b],
            in_sems[b],
        )

    def out_copy(g, b):
        m0 = base + g * _G
        return pltpu.make_async_copy(
            out_bufs[b],
            out_hbm.at[pl.ds(m0 * _NTRIL, _G * _NTRIL)],
            out_sems[b],
        )

    in_copy(0, 0).start()
    in_copy(1, 1).start()

    def outer(it, carry):
        for b in range(2):
            g = it * 2 + b
            in_copy(g, b).wait()

            @pl.when(g >= 2)
            def _():
                out_copy(g - 2, b).wait()

            @plsc.parallel_loop(0, _VECS, unroll=8)
            def compact(k):
                idx = idx_v[pl.ds(k * _LANES, _LANES)]
                out_bufs[b][pl.ds(k * _LANES, _LANES)] = plsc.load_gather(
                    in_bufs[b], [idx])

            out_copy(g, b).start()

            @pl.when(g + 2 < _CHUNKS)
            def _():
                in_copy(g + 2, b).start()

        return carry

    lax.fori_loop(0, _CHUNKS // 2, outer, 0)
    out_copy(_CHUNKS - 2, 0).wait()
    out_copy(_CHUNKS - 1, 1).wait()


@jax.jit
def _spd_vectorize(flat_in, idx):
    mesh = plsc.VectorSubcoreMesh(core_axis_name="c", subcore_axis_name="s")
    return pl.kernel(
        _body,
        mesh=mesh,
        out_type=jax.ShapeDtypeStruct((_M * _NTRIL,), jnp.float32),
        scratch_types=[
            pltpu.VMEM((_G * _NTRIL,), jnp.int32),
            pltpu.VMEM((_G * _C * _C,), jnp.float32),
            pltpu.VMEM((_G * _C * _C,), jnp.float32),
            pltpu.VMEM((_G * _NTRIL,), jnp.float32),
            pltpu.VMEM((_G * _NTRIL,), jnp.float32),
            pltpu.SemaphoreType.DMA,
            pltpu.SemaphoreType.DMA,
            pltpu.SemaphoreType.DMA,
            pltpu.SemaphoreType.DMA,
        ],
        compiler_params=pltpu.CompilerParams(needs_layout_passes=False),
    )(flat_in, idx)


def kernel(inputs):
    flat = inputs.reshape(-1)
    idx = jnp.asarray(_IDX_HOST)
    out = _spd_vectorize(flat, idx)
    return out.reshape(256, 32, 8, _NTRIL)


# speedup vs baseline: 17.8597x; 12.1920x over previous
import jax
import jax.numpy as jnp
import numpy as np
from jax import lax
from jax.experimental import pallas as pl
from jax.experimental.pallas import tpu as pltpu
from jax.experimental.pallas import tpu_sc as plsc

_C = 32
_NTRIL = _C * (_C + 1) // 2
_T = 256
_NB = 32 * 8
_D = 128

_ROWS_IN = _NB * _C * 4 * 2 * 8
_ROWS_OUT = _NB * 66 * 2 * 8

_NC = 2
_NS = 16
_NW = _NC * _NS
_RPW = _ROWS_OUT // _NW
_CH = 128
_NCH = _RPW // _CH
_NBUF = 6


def _build_idx() -> np.ndarray:
    row, col = np.tril_indices(_C)
    base = (row * 4 + col // 8) * 16 + col % 8
    bk = base.reshape(66, 8)
    nb = np.arange(_NB, dtype=np.int32)
    tt = np.arange(2, dtype=np.int32)
    j = (nb[:, None, None, None] * 2048
         + bk[None, :, None, :]
         + tt[None, None, :, None] * 8)
    return j.reshape(_ROWS_OUT).astype(np.int32)


_IDX_HOST = _build_idx()


def _body(tab_hbm, idx_hbm, out_hbm, idx_v, bufs, gsems, ssems):
    wid = lax.axis_index("s") * _NC + lax.axis_index("c")
    r0 = wid * _RPW
    pltpu.sync_copy(idx_hbm.at[pl.ds(r0, _RPW)], idx_v)

    def gather(g, b):
        return pltpu.make_async_copy(
            tab_hbm.at[idx_v.at[pl.ds(g * _CH, _CH)]], bufs[b], gsems[b])

    def scatter(g, b):
        return pltpu.make_async_copy(
            bufs[b], out_hbm.at[pl.ds(r0 + g * _CH, _CH)], ssems[b])

    gather(0, 0).start()
    gather(1, 1).start()

    def outer(it, carry):
        for b in range(_NBUF):
            g = it * _NBUF + b
            gather(g, b).wait()
            scatter(g, b).start()
            n = g + 2
            bn = (b + 2) % _NBUF

            @pl.when(n < _NCH)
            def _():
                @pl.when(n >= _NBUF)
                def _():
                    scatter(n - _NBUF, bn).wait()

                gather(n, bn).start()

        return carry

    lax.fori_loop(0, _NCH // _NBUF, outer, 0)
    for b in range(_NBUF):
        scatter(_NCH - _NBUF + b, b).wait()


@jax.jit
def _spd_vectorize(tab, idx):
    mesh = plsc.VectorSubcoreMesh(core_axis_name="c", subcore_axis_name="s")
    return pl.kernel(
        _body,
        mesh=mesh,
        out_type=jax.ShapeDtypeStruct((_ROWS_OUT, _D), jnp.float32),
        scratch_types={
            "idx_v": pltpu.VMEM((_RPW,), jnp.int32),
            "bufs": [pltpu.VMEM((_CH, _D), jnp.float32)] * _NBUF,
            "gsems": [pltpu.SemaphoreType.DMA] * _NBUF,
            "ssems": [pltpu.SemaphoreType.DMA] * _NBUF,
        },
    )(tab, idx)


def kernel(inputs):
    x = jnp.transpose(inputs, (1, 2, 3, 4, 0))
    x = x.reshape(32, 8, _C, 4, 8, 2, _D)
    x = jnp.transpose(x, (0, 1, 2, 3, 5, 4, 6))
    tab = x.reshape(_ROWS_IN, _D)

    y = _spd_vectorize(tab, jnp.asarray(_IDX_HOST))

    y = y.reshape(32, 8, 66, 2, 8, _D)
    y = jnp.transpose(y, (3, 5, 0, 1, 2, 4))
    return y.reshape(_T, 32, 8, _NTRIL)
